# Initial kernel scaffold; baseline (speedup 1.0000x reference)
import jax
import jax.numpy as jnp
from jax.experimental import pallas as pl


def kernel(U_t, U_a, U_v, lengths, qmast, e, i, params, edge_index):
    raise NotImplementedError("write your pallas kernel here")


# speedup vs baseline: 15.6394x; 15.6394x over previous
import numpy as np
import jax
import jax.numpy as jnp
from jax.experimental import pallas as pl
from jax.experimental.pallas import tpu as pltpu

_L, _B, _NSPK = 100, 128, 9
_DT = 1024
_DG = 128
_H = 64
_NCLS = 7
_WIN = 16
_N = _L * _B

_STAT_TILE = 256
_BT = 16
_CH = 10


def _stats_body(u_ref, s_ref, q_ref):
    i = pl.program_id(0)
    x = u_ref[...]
    ss, qq = [], []
    for j in range(4):
        xj = x[j]
        ss.append(jnp.sum(xj, axis=0, keepdims=True))
        qq.append(jnp.sum(xj * xj, axis=0, keepdims=True))
    s = jnp.concatenate(ss, 0)
    q = jnp.concatenate(qq, 0)

    @pl.when(i == 0)
    def _():
        s_ref[...] = jnp.zeros_like(s_ref)
        q_ref[...] = jnp.zeros_like(q_ref)

    s_ref[...] += s
    q_ref[...] += q


def _bnproj_body(u_ref, s_ref, q_ref, g_ref, b_ref, wt_ref, bt_ref, o_ref):
    inv_n = 1.0 / _N
    mu = s_ref[...] * inv_n
    var = q_ref[...] * inv_n - mu * mu
    a = g_ref[...] * jax.lax.rsqrt(var + 1e-5)
    c = b_ref[...] - mu * a
    x = u_ref[...]
    acc = x[0] * a[0:1]
    for j in range(1, 4):
        acc = acc + x[j] * a[j:j + 1]
    wt = wt_ref[...]
    y = jnp.dot(acc * 0.25, wt, preferred_element_type=jnp.float32)
    cbar = jnp.sum(c, axis=0, keepdims=True) * 0.25
    y = y + jnp.dot(cbar, wt, preferred_element_type=jnp.float32)
    o_ref[...] = y + bt_ref[...]


def _gru_body(x_ref, wf1, wb1, bx1, wh1, bh1, wf2, wb2, bx2, wh2, bh2,
              o_ref, gxs, ysf, ysb):
    nch = _L // _CH

    def run_layer(read_row, wf, wb, bx, wh, bh, write_out):
        wfv = wf[...]
        wbv = wb[...]
        bxv = bx[...]
        for cidx in range(nch):
            xc = jnp.concatenate(
                [read_row(cidx * _CH + k) for k in range(_CH)], axis=0)
            gf = jnp.dot(xc, wfv, preferred_element_type=jnp.float32) + bxv
            gxs[pl.ds(cidx * _CH, _CH)] = gf.reshape(_CH, _B, 384)
        for cidx in range(nch):
            xr = jnp.concatenate(
                [read_row(_L - 1 - cidx * _CH - k) for k in range(_CH)], axis=0)
            gb = jnp.dot(xr, wbv, preferred_element_type=jnp.float32)
            gxs[pl.ds(cidx * _CH, _CH)] += gb.reshape(_CH, _B, 384)
        whv = wh[...]
        bhv = bh[...]

        def step(t, h):
            gx = gxs[t]
            gh = jnp.dot(h, whv, preferred_element_type=jnp.float32) + bhv
            rz = jax.nn.sigmoid(gx[:, :256] + gh[:, :256])
            r = rz[:, :128]
            z = rz[:, 128:]
            n = jnp.tanh(gx[:, 256:] + r * gh[:, 256:])
            hn = (1.0 - z) * n + z * h
            ysf[t] = hn
            ysb[_L - 1 - t] = hn
            return hn

        jax.lax.fori_loop(0, _L, step, jnp.zeros((_B, _DG), jnp.float32))
        lane = jax.lax.broadcasted_iota(jnp.int32, (25, _B, _DG), 2)
        for cidx in range(4):
            sl = pl.ds(cidx * 25, 25)
            write_out(sl, jnp.where(lane < _H, ysf[sl], ysb[sl]))

    def read1(t):
        return x_ref[t]

    lane2 = jax.lax.broadcasted_iota(jnp.int32, (_B, _DG), 1)

    def read2(t):
        return jnp.where(lane2 < _H, ysf[t], ysb[t])

    def write2(sl, blk):
        o_ref[sl] = blk

    run_layer(read1, wf1, wb1, bx1, wh1, bh1, lambda sl, blk: None)
    run_layer(read2, wf2, wb2, bx2, wh2, bh2, write2)


def _head_body(ua_ref, uv_ref, ft_ref, qm_ref, an_ref,
               wa, ba, wv, bv, wspk, wg1, bg1, wg2, bg2,
               wfc1, bfc1, wfc2, bfc2, emo_ref, lp_ref):
    lbt = _L * _BT
    an = an_ref[...]
    spk = jnp.dot(qm_ref[...].reshape(lbt, _DG), wspk[...],
                  preferred_element_type=jnp.float32)
    xa = jnp.dot(ua_ref[...].reshape(lbt, 300), wa[...],
                 preferred_element_type=jnp.float32) + ba[...] + spk
    xv = jnp.dot(uv_ref[...].reshape(lbt, 342), wv[...],
                 preferred_element_type=jnp.float32) + bv[...] + spk
    xt = ft_ref[...].reshape(lbt, _DG) + spk
    w1 = wg1[...]
    w2 = wg2[...]
    b1 = bg1[...]
    b2 = bg2[...]

    def gcn(x):
        h1p = jnp.dot(an, x.reshape(_L, _BT * _DG),
                      preferred_element_type=jnp.float32).reshape(lbt, _DG)
        h1 = jax.nn.relu(jnp.dot(h1p, w1, preferred_element_type=jnp.float32)
                         + b1) + x
        h2p = jnp.dot(an, h1.reshape(_L, _BT * _DG),
                      preferred_element_type=jnp.float32).reshape(lbt, _DG)
        h2 = jax.nn.relu(jnp.dot(h2p, w2, preferred_element_type=jnp.float32)
                         + b2) + h1
        return x, h1, h2

    parts = []
    for x in (xa, xv, xt):
        parts.extend(gcn(x))
    emo = jnp.concatenate([p.reshape(_L, _BT, _DG) for p in parts], axis=2)
    emo_ref[...] = jnp.transpose(emo, (1, 0, 2))

    ef = jax.nn.relu(emo).reshape(lbt, 9 * _DG)
    l1 = jnp.dot(ef, wfc1[...], preferred_element_type=jnp.float32) + bfc1[...]
    l2 = jnp.dot(l1, wfc2[...], preferred_element_type=jnp.float32) + bfc2[...]
    m = jnp.max(l2, axis=1, keepdims=True)
    lse = m + jnp.log(jnp.sum(jnp.exp(l2 - m), axis=1, keepdims=True))
    lp = jnp.pad(l2 - lse, ((0, 0), (0, _DG - _NCLS)))
    lp_ref[...] = jnp.transpose(lp.reshape(_L, _BT, _DG), (1, 0, 2))


def _prep_gru_weights(p, layer):
    wf = p['gru_Wih_l%d_d0' % layer]
    wb = p['gru_Wih_l%d_d1' % layer]
    whf = p['gru_Whh_l%d_d0' % layer]
    whb = p['gru_Whh_l%d_d1' % layer]
    bf = p['gru_bih_l%d_d0' % layer]
    bb = p['gru_bih_l%d_d1' % layer]
    bhf = p['gru_bhh_l%d_d0' % layer]
    bhb = p['gru_bhh_l%d_d1' % layer]
    WF = jnp.zeros((_DG, 384), jnp.float32)
    WB = jnp.zeros((_DG, 384), jnp.float32)
    WH = jnp.zeros((_DG, 384), jnp.float32)
    BX = jnp.zeros((384,), jnp.float32)
    BH = jnp.zeros((384,), jnp.float32)
    for g in range(3):
        gs = slice(g * _H, (g + 1) * _H)
        fcol = slice(g * _DG, g * _DG + _H)
        bcol = slice(g * _DG + _H, (g + 1) * _DG)
        WF = WF.at[:, fcol].set(wf[gs, :].T)
        WB = WB.at[:, bcol].set(wb[gs, :].T)
        WH = WH.at[:_H, fcol].set(whf[gs, :].T)
        WH = WH.at[_H:, bcol].set(whb[gs, :].T)
        BX = BX.at[fcol].set(bf[gs]).at[bcol].set(bb[gs])
        BH = BH.at[fcol].set(bhf[gs]).at[bcol].set(bhb[gs])
    return WF, WB, BX.reshape(1, 384), WH, BH.reshape(1, 384)


def _band_matrix():
    deg = np.minimum(np.arange(_L) + 1, _WIN + 1).astype(np.float32)
    an = np.zeros((_L, _L), np.float32)
    for t in range(_L):
        an[t, max(0, t - _WIN):t + 1] = 1.0 / deg[t]
    return jnp.asarray(an)


def kernel(U_t, U_a, U_v, lengths, qmast, e, i, params, edge_index):
    p = params
    f32 = jnp.float32
    u4 = U_t.reshape(4, _N, _DT)
    n_tiles = _N // _STAT_TILE

    sums, sqs = pl.pallas_call(
        _stats_body,
        grid=(n_tiles,),
        in_specs=[pl.BlockSpec((4, _STAT_TILE, _DT), lambda k: (0, k, 0))],
        out_specs=[pl.BlockSpec((4, _DT), lambda k: (0, 0)),
                   pl.BlockSpec((4, _DT), lambda k: (0, 0))],
        out_shape=[jax.ShapeDtypeStruct((4, _DT), f32)] * 2,
    )(u4)

    ut_proj = pl.pallas_call(
        _bnproj_body,
        grid=(n_tiles,),
        in_specs=[
            pl.BlockSpec((4, _STAT_TILE, _DT), lambda k: (0, k, 0)),
            pl.BlockSpec((4, _DT), lambda k: (0, 0)),
            pl.BlockSpec((4, _DT), lambda k: (0, 0)),
            pl.BlockSpec((4, _DT), lambda k: (0, 0)),
            pl.BlockSpec((4, _DT), lambda k: (0, 0)),
            pl.BlockSpec((_DT, _DG), lambda k: (0, 0)),
            pl.BlockSpec((1, _DG), lambda k: (0, 0)),
        ],
        out_specs=pl.BlockSpec((_STAT_TILE, _DG), lambda k: (k, 0)),
        out_shape=jax.ShapeDtypeStruct((_N, _DG), f32),
    )(u4, sums, sqs, p['bn_gamma'], p['bn_beta'], p['W_t'],
      p['b_t'].reshape(1, _DG))

    g1 = _prep_gru_weights(p, 0)
    g2 = _prep_gru_weights(p, 1)
    ft = pl.pallas_call(
        _gru_body,
        out_shape=jax.ShapeDtypeStruct((_L, _B, _DG), f32),
        scratch_shapes=[
            pltpu.VMEM((_L, _B, 384), f32),
            pltpu.VMEM((_L, _B, _DG), f32),
            pltpu.VMEM((_L, _B, _DG), f32),
        ],
    )(ut_proj.reshape(_L, _B, _DG), *g1, *g2)

    qm = jnp.pad(qmast, ((0, 0), (0, 0), (0, _DG - _NSPK)))
    wspk = jnp.pad(p['W_spk'], ((0, _DG - _NSPK), (0, 0)))
    an = _band_matrix()
    const = lambda shape: pl.BlockSpec(shape, lambda j: tuple(0 for _ in shape))
    emo_b, lp_b = pl.pallas_call(
        _head_body,
        grid=(_B // _BT,),
        in_specs=[
            pl.BlockSpec((_L, _BT, 300), lambda j: (0, j, 0)),
            pl.BlockSpec((_L, _BT, 342), lambda j: (0, j, 0)),
            pl.BlockSpec((_L, _BT, _DG), lambda j: (0, j, 0)),
            pl.BlockSpec((_L, _BT, _DG), lambda j: (0, j, 0)),
            const((_L, _L)),
            const((300, _DG)), const((1, _DG)),
            const((342, _DG)), const((1, _DG)),
            const((_DG, _DG)),
            const((_DG, _DG)), const((1, _DG)),
            const((_DG, _DG)), const((1, _DG)),
            const((9 * _DG, 84)), const((1, 84)),
            const((84, _NCLS)), const((1, _NCLS)),
        ],
        out_specs=[pl.BlockSpec((_BT, _L, 9 * _DG), lambda j: (j, 0, 0)),
                   pl.BlockSpec((_BT, _L, _DG), lambda j: (j, 0, 0))],
        out_shape=[jax.ShapeDtypeStruct((_B, _L, 9 * _DG), f32),
                   jax.ShapeDtypeStruct((_B, _L, _DG), f32)],
    )(U_a, U_v, ft, qm, an,
      p['W_a'], p['b_a'].reshape(1, _DG),
      p['W_v'], p['b_v'].reshape(1, _DG),
      wspk,
      p['Wg1'], p['bg1'].reshape(1, _DG),
      p['Wg2'], p['bg2'].reshape(1, _DG),
      p['W_fc1'], p['b_fc1'].reshape(1, 84),
      p['W_fc2'], p['b_fc2'].reshape(1, _NCLS))

    emotions_feat = emo_b.reshape(_N, 9 * _DG)
    log_prob = lp_b.reshape(_N, _DG)[:, :_NCLS]
    zero = jnp.zeros((), f32)
    return (log_prob, zero, zero, emotions_feat)


# speedup vs baseline: 17.1566x; 1.0970x over previous
import numpy as np
import jax
import jax.numpy as jnp
from jax.experimental import pallas as pl
from jax.experimental.pallas import tpu as pltpu

_L, _B, _NSPK = 100, 128, 9
_DT = 1024
_DG = 128
_H = 64
_NCLS = 7
_WIN = 16
_N = _L * _B

_STAT_TILE = 512
_BT = 16
_CH = 10


def _stats_body(u_ref, s_ref, q_ref):
    i = pl.program_id(0)
    x = u_ref[...]
    ss, qq = [], []
    for j in range(4):
        xj = x[j]
        ss.append(jnp.sum(xj, axis=0, keepdims=True))
        qq.append(jnp.sum(xj * xj, axis=0, keepdims=True))
    s = jnp.concatenate(ss, 0)
    q = jnp.concatenate(qq, 0)

    @pl.when(i == 0)
    def _():
        s_ref[...] = jnp.zeros_like(s_ref)
        q_ref[...] = jnp.zeros_like(q_ref)

    s_ref[...] += s
    q_ref[...] += q


def _bnproj_body(u_ref, s_ref, q_ref, g_ref, b_ref, wt_ref, bt_ref, o_ref):
    inv_n = 1.0 / _N
    mu = s_ref[...] * inv_n
    var = q_ref[...] * inv_n - mu * mu
    a = g_ref[...] * jax.lax.rsqrt(var + 1e-5)
    c = b_ref[...] - mu * a
    x = u_ref[...]
    acc = x[0] * a[0:1]
    for j in range(1, 4):
        acc = acc + x[j] * a[j:j + 1]
    wt = wt_ref[...]
    y = jnp.dot(acc * 0.25, wt, preferred_element_type=jnp.float32)
    cbar = jnp.sum(c, axis=0, keepdims=True) * 0.25
    y = y + jnp.dot(cbar, wt, preferred_element_type=jnp.float32)
    o_ref[...] = y + bt_ref[...]


def _gru_body(x_ref, wf1, wb1, bx1, wh1, bh1, wf2, wb2, bx2, wh2, bh2,
              o_ref, gxs, ysf, ysb):
    nch = _L // _CH

    def run_layer(read_row, wf, wb, bx, wh, bh, write_out):
        wfv = wf[...]
        wbv = wb[...]
        bxv = bx[...]
        for cidx in range(nch):
            xc = jnp.concatenate(
                [read_row(cidx * _CH + k) for k in range(_CH)], axis=0)
            gf = jnp.dot(xc, wfv, preferred_element_type=jnp.float32) + bxv
            gxs[pl.ds(cidx * _CH, _CH)] = gf.reshape(_CH, _B, 384)
        for cidx in range(nch):
            xr = jnp.concatenate(
                [read_row(_L - 1 - cidx * _CH - k) for k in range(_CH)], axis=0)
            gb = jnp.dot(xr, wbv, preferred_element_type=jnp.float32)
            gxs[pl.ds(cidx * _CH, _CH)] += gb.reshape(_CH, _B, 384)
        whv = wh[...]
        bhv = bh[...]

        def step(t, h):
            gx = gxs[t]
            gh = jnp.dot(h, whv, preferred_element_type=jnp.float32) + bhv
            rz = jax.nn.sigmoid(gx[:, :256] + gh[:, :256])
            r = rz[:, :128]
            z = rz[:, 128:]
            n = jnp.tanh(gx[:, 256:] + r * gh[:, 256:])
            hn = (1.0 - z) * n + z * h
            ysf[t] = hn
            ysb[_L - 1 - t] = hn
            return hn

        jax.lax.fori_loop(0, _L, step, jnp.zeros((_B, _DG), jnp.float32))
        lane = jax.lax.broadcasted_iota(jnp.int32, (25, _B, _DG), 2)
        for cidx in range(4):
            sl = pl.ds(cidx * 25, 25)
            write_out(sl, jnp.where(lane < _H, ysf[sl], ysb[sl]))

    def read1(t):
        return x_ref[t]

    lane2 = jax.lax.broadcasted_iota(jnp.int32, (_B, _DG), 1)

    def read2(t):
        return jnp.where(lane2 < _H, ysf[t], ysb[t])

    def write2(sl, blk):
        o_ref[sl] = blk

    run_layer(read1, wf1, wb1, bx1, wh1, bh1, lambda sl, blk: None)
    run_layer(read2, wf2, wb2, bx2, wh2, bh2, write2)


def _head_body(ua_ref, uv_ref, ft_ref, qm_ref, an_ref,
               wa, ba, wv, bv, wspk, wg1, bg1, wg2, bg2,
               wfc1, bfc1, wfc2, bfc2, emo_ref, lp_ref):
    lbt = _L * _BT
    an = an_ref[...]
    spk = jnp.dot(qm_ref[...].reshape(lbt, _NSPK), wspk[...],
                  preferred_element_type=jnp.float32)
    xa = jnp.dot(ua_ref[...].reshape(lbt, 300), wa[...],
                 preferred_element_type=jnp.float32) + ba[...] + spk
    xv = jnp.dot(uv_ref[...].reshape(lbt, 342), wv[...],
                 preferred_element_type=jnp.float32) + bv[...] + spk
    xt = ft_ref[...].reshape(lbt, _DG) + spk
    w1 = wg1[...]
    w2 = wg2[...]
    b1 = bg1[...]
    b2 = bg2[...]

    def gcn(x):
        h1p = jnp.dot(an, x.reshape(_L, _BT * _DG),
                      preferred_element_type=jnp.float32).reshape(lbt, _DG)
        h1 = jax.nn.relu(jnp.dot(h1p, w1, preferred_element_type=jnp.float32)
                         + b1) + x
        h2p = jnp.dot(an, h1.reshape(_L, _BT * _DG),
                      preferred_element_type=jnp.float32).reshape(lbt, _DG)
        h2 = jax.nn.relu(jnp.dot(h2p, w2, preferred_element_type=jnp.float32)
                         + b2) + h1
        return x, h1, h2

    parts = []
    for x in (xa, xv, xt):
        parts.extend(gcn(x))
    emo = jnp.concatenate([p.reshape(_L, _BT, _DG) for p in parts], axis=2)
    emo_ref[...] = jnp.transpose(emo, (1, 0, 2))

    ef = jax.nn.relu(emo).reshape(lbt, 9 * _DG)
    l1 = jnp.dot(ef, wfc1[...], preferred_element_type=jnp.float32) + bfc1[...]
    l2 = jnp.dot(l1, wfc2[...], preferred_element_type=jnp.float32) + bfc2[...]
    m = jnp.max(l2, axis=1, keepdims=True)
    lse = m + jnp.log(jnp.sum(jnp.exp(l2 - m), axis=1, keepdims=True))
    lp = jnp.pad(l2 - lse, ((0, 0), (0, 8 - _NCLS)))
    lp_ref[...] = jnp.transpose(lp.reshape(_L, _BT, 8), (1, 0, 2))


def _prep_gru_weights(p, layer):
    wf = p['gru_Wih_l%d_d0' % layer]
    wb = p['gru_Wih_l%d_d1' % layer]
    whf = p['gru_Whh_l%d_d0' % layer]
    whb = p['gru_Whh_l%d_d1' % layer]
    bf = p['gru_bih_l%d_d0' % layer]
    bb = p['gru_bih_l%d_d1' % layer]
    bhf = p['gru_bhh_l%d_d0' % layer]
    bhb = p['gru_bhh_l%d_d1' % layer]
    WF = jnp.zeros((_DG, 384), jnp.float32)
    WB = jnp.zeros((_DG, 384), jnp.float32)
    WH = jnp.zeros((_DG, 384), jnp.float32)
    BX = jnp.zeros((384,), jnp.float32)
    BH = jnp.zeros((384,), jnp.float32)
    for g in range(3):
        gs = slice(g * _H, (g + 1) * _H)
        fcol = slice(g * _DG, g * _DG + _H)
        bcol = slice(g * _DG + _H, (g + 1) * _DG)
        WF = WF.at[:, fcol].set(wf[gs, :].T)
        WB = WB.at[:, bcol].set(wb[gs, :].T)
        WH = WH.at[:_H, fcol].set(whf[gs, :].T)
        WH = WH.at[_H:, bcol].set(whb[gs, :].T)
        BX = BX.at[fcol].set(bf[gs]).at[bcol].set(bb[gs])
        BH = BH.at[fcol].set(bhf[gs]).at[bcol].set(bhb[gs])
    return WF, WB, BX.reshape(1, 384), WH, BH.reshape(1, 384)


def _band_matrix():
    deg = np.minimum(np.arange(_L) + 1, _WIN + 1).astype(np.float32)
    an = np.zeros((_L, _L), np.float32)
    for t in range(_L):
        an[t, max(0, t - _WIN):t + 1] = 1.0 / deg[t]
    return jnp.asarray(an)


def kernel(U_t, U_a, U_v, lengths, qmast, e, i, params, edge_index):
    p = params
    f32 = jnp.float32
    u4 = U_t.reshape(4, _N, _DT)
    n_tiles = _N // _STAT_TILE

    sums, sqs = pl.pallas_call(
        _stats_body,
        grid=(n_tiles,),
        in_specs=[pl.BlockSpec((4, _STAT_TILE, _DT), lambda k: (0, k, 0))],
        out_specs=[pl.BlockSpec((4, _DT), lambda k: (0, 0)),
                   pl.BlockSpec((4, _DT), lambda k: (0, 0))],
        out_shape=[jax.ShapeDtypeStruct((4, _DT), f32)] * 2,
    )(u4)

    ut_proj = pl.pallas_call(
        _bnproj_body,
        grid=(n_tiles,),
        in_specs=[
            pl.BlockSpec((4, _STAT_TILE, _DT), lambda k: (0, k, 0)),
            pl.BlockSpec((4, _DT), lambda k: (0, 0)),
            pl.BlockSpec((4, _DT), lambda k: (0, 0)),
            pl.BlockSpec((4, _DT), lambda k: (0, 0)),
            pl.BlockSpec((4, _DT), lambda k: (0, 0)),
            pl.BlockSpec((_DT, _DG), lambda k: (0, 0)),
            pl.BlockSpec((1, _DG), lambda k: (0, 0)),
        ],
        out_specs=pl.BlockSpec((_STAT_TILE, _DG), lambda k: (k, 0)),
        out_shape=jax.ShapeDtypeStruct((_N, _DG), f32),
    )(u4, sums, sqs, p['bn_gamma'], p['bn_beta'], p['W_t'],
      p['b_t'].reshape(1, _DG))

    g1 = _prep_gru_weights(p, 0)
    g2 = _prep_gru_weights(p, 1)
    ft = pl.pallas_call(
        _gru_body,
        out_shape=jax.ShapeDtypeStruct((_L, _B, _DG), f32),
        scratch_shapes=[
            pltpu.VMEM((_L, _B, 384), f32),
            pltpu.VMEM((_L, _B, _DG), f32),
            pltpu.VMEM((_L, _B, _DG), f32),
        ],
    )(ut_proj.reshape(_L, _B, _DG), *g1, *g2)

    an = _band_matrix()
    const = lambda shape: pl.BlockSpec(shape, lambda j: tuple(0 for _ in shape))
    emo_b, lp_b = pl.pallas_call(
        _head_body,
        grid=(_B // _BT,),
        in_specs=[
            pl.BlockSpec((_L, _BT, 300), lambda j: (0, j, 0)),
            pl.BlockSpec((_L, _BT, 342), lambda j: (0, j, 0)),
            pl.BlockSpec((_L, _BT, _DG), lambda j: (0, j, 0)),
            pl.BlockSpec((_L, _BT, _NSPK), lambda j: (0, j, 0)),
            const((_L, _L)),
            const((300, _DG)), const((1, _DG)),
            const((342, _DG)), const((1, _DG)),
            const((_NSPK, _DG)),
            const((_DG, _DG)), const((1, _DG)),
            const((_DG, _DG)), const((1, _DG)),
            const((9 * _DG, 84)), const((1, 84)),
            const((84, _NCLS)), const((1, _NCLS)),
        ],
        out_specs=[pl.BlockSpec((_BT, _L, 9 * _DG), lambda j: (j, 0, 0)),
                   pl.BlockSpec((_BT, _L, 8), lambda j: (j, 0, 0))],
        out_shape=[jax.ShapeDtypeStruct((_B, _L, 9 * _DG), f32),
                   jax.ShapeDtypeStruct((_B, _L, 8), f32)],
    )(U_a, U_v, ft, qmast, an,
      p['W_a'], p['b_a'].reshape(1, _DG),
      p['W_v'], p['b_v'].reshape(1, _DG),
      p['W_spk'],
      p['Wg1'], p['bg1'].reshape(1, _DG),
      p['Wg2'], p['bg2'].reshape(1, _DG),
      p['W_fc1'], p['b_fc1'].reshape(1, 84),
      p['W_fc2'], p['b_fc2'].reshape(1, _NCLS))

    emotions_feat = emo_b.reshape(_N, 9 * _DG)
    log_prob = lp_b.reshape(_N, 8)[:, :_NCLS]
    zero = jnp.zeros((), f32)
    return (log_prob, zero, zero, emotions_feat)
